# baseline (speedup 1.0000x reference)
import functools

import jax
import jax.numpy as jnp
from jax import lax
from jax.experimental import pallas as pl
from jax.experimental.pallas import tpu as pltpu
from jax.experimental.pallas import tpu_sc as plsc

N = 10000
DH = 128
E = 320000
NSTEPS = 20
DT = 0.9 / NSTEPS

NT = 16
K = 128
NCH = 160
EPT = NCH * K
EPAD = NT * EPT
NPAD = 10240
RPT = NPAD // NT
PK = 2 * K
TOT = E + NT * PK
TOTCH = TOT // K


def _spmm_body(sa, sb, edg, meta, axa, axb,
               acc, ebuf0, ebuf1, rows0, rows1, metab,
               esem0, esem1, gsem0, gsem1):
    c = lax.axis_index("c")
    s = lax.axis_index("s")
    ebuf = (ebuf0, ebuf1)
    rows = (rows0, rows1)
    esem = (esem0, esem1)
    gsem = (gsem0, gsem1)
    base = s * RPT

    pltpu.sync_copy(meta, metab)
    mv = metab[s, pl.ds(0, 16)]
    n = mv[0]
    c0 = mv[1]

    def zacc(r, carry):
        for l in range(DH // 16):
            acc[r, pl.ds(l * 16, 16)] = jnp.zeros((16,), jnp.float32)
        return carry
    lax.fori_loop(0, RPT, zacc, 0)

    def start_stage(it, q):
        ch = pl.multiple_of((c0 + it) * 8, 8)
        pltpu.async_copy(edg.at[pl.ds(ch, 8)], ebuf[q], esem[q])

    def wait_stage(q):
        pltpu.make_async_copy(edg.at[pl.ds(0, 8)], ebuf[q], esem[q]).wait()

    def dadj(q):
        for l in range(K // 16):
            ebuf[q][3, pl.ds(l * 16, 16)] = (
                ebuf[q][1, pl.ds(l * 16, 16)] - base)

    def start_gather(q):
        @pl.when(c == 0)
        def _():
            pltpu.async_copy(sa.at[ebuf[q].at[0]], rows[q], gsem[q])

        @pl.when(c == 1)
        def _():
            pltpu.async_copy(sb.at[ebuf[q].at[0]], rows[q], gsem[q])

    def wait_gather(q):
        pltpu.make_async_copy(sa.at[ebuf[0].at[0]], rows[q], gsem[q]).wait()

    def compute(q):
        rq = rows[q]

        def body(jo, cc):
            wv = lax.bitcast_convert_type(
                ebuf[q][2, pl.ds(jo * 16, 16)], jnp.float32)
            dv = ebuf[q][3, pl.ds(jo * 16, 16)]
            for j in range(16):
                e = jo * 16 + j
                wsc = wv[j]
                d = dv[j]
                for l in range(DH // 16):
                    plsc.addupdate(acc.at[d, pl.ds(l * 16, 16)],
                                   rq[e, pl.ds(l * 16, 16)] * wsc)
            return cc
        lax.fori_loop(0, K // 16, body, 0)

    @pl.when(n > 0)
    def _():
        start_stage(0, 0)
        start_stage(1, 1)
        wait_stage(0)
        dadj(0)
        start_gather(0)

    def pair(p, carry):
        for q in range(2):
            it = 2 * p + q
            q2 = (q + 1) % 2
            wait_gather(q)

            @pl.when(it + 1 < n)
            def _():
                wait_stage(q2)
                dadj(q2)
                start_gather(q2)
            compute(q)

            @pl.when(it + 2 < n)
            def _():
                start_stage(it + 2, q)
        return carry
    lax.fori_loop(0, n // 2, pair, 0)

    @pl.when(c == 0)
    def _():
        pltpu.sync_copy(acc, axa.at[pl.ds(base, RPT)])

    @pl.when(c == 1)
    def _():
        pltpu.sync_copy(acc, axb.at[pl.ds(base, RPT)])


_spmm = functools.partial(
    pl.kernel,
    out_type=(jax.ShapeDtypeStruct((NPAD, DH), jnp.float32),
              jax.ShapeDtypeStruct((NPAD, DH), jnp.float32)),
    mesh=plsc.VectorSubcoreMesh(core_axis_name="c", subcore_axis_name="s"),
    scratch_types=[
        pltpu.VMEM((RPT, DH), jnp.float32),
        pltpu.VMEM((8, K), jnp.int32),
        pltpu.VMEM((8, K), jnp.int32),
        pltpu.VMEM((K, DH), jnp.float32),
        pltpu.VMEM((K, DH), jnp.float32),
        pltpu.VMEM((16, 16), jnp.int32),
        pltpu.SemaphoreType.DMA,
        pltpu.SemaphoreType.DMA,
        pltpu.SemaphoreType.DMA,
        pltpu.SemaphoreType.DMA,
    ],
)(_spmm_body)


BN = 1024
GRID = NPAD // BN


def _update_body(sa, sb, axa, axb, xr, al, hr, wih, whh, bih, bhh, wmat,
                 osa, osb, oal):
    st = jnp.concatenate([sa[...], sb[...]], axis=1)
    z = jnp.dot(st, wih[...].T, preferred_element_type=jnp.float32)
    z = z + jnp.dot(hr[...], whh[...].T, preferred_element_type=jnp.float32)
    z = z + bih[...] + bhh[...]
    r = jnp.tanh(z)
    alpha_new = al[...] * r[:, 0:1] + r[:, 1:2]
    alph = jax.nn.sigmoid(alpha_new)
    xw = jnp.dot(st, wmat[...], preferred_element_type=jnp.float32)
    ax = jnp.concatenate([axa[...], axb[...]], axis=1)
    xv = xr[...]
    x0 = jnp.concatenate([xv, jnp.zeros_like(xv)], axis=1)
    f = alph * 0.5 * (ax - st) + xw - st + x0
    st2 = st + DT * f
    osa[...] = st2[:, :DH]
    osb[...] = st2[:, DH:]
    oal[...] = alpha_new


def _update(sa, sb, axa, axb, x, alpha, h, wih, whh, bih, bhh, wmat):
    row_spec = pl.BlockSpec((BN, DH), lambda i: (i, 0))
    return pl.pallas_call(
        _update_body,
        grid=(GRID,),
        in_specs=[
            row_spec, row_spec, row_spec, row_spec, row_spec,
            pl.BlockSpec((BN, 1), lambda i: (i, 0)),
            pl.BlockSpec((BN, 2), lambda i: (i, 0)),
            pl.BlockSpec((2, 2 * DH), lambda i: (0, 0)),
            pl.BlockSpec((2, 2), lambda i: (0, 0)),
            pl.BlockSpec((1, 2), lambda i: (0, 0)),
            pl.BlockSpec((1, 2), lambda i: (0, 0)),
            pl.BlockSpec((2 * DH, 2 * DH), lambda i: (0, 0)),
        ],
        out_specs=[
            row_spec, row_spec,
            pl.BlockSpec((BN, 1), lambda i: (i, 0)),
        ],
        out_shape=[
            jax.ShapeDtypeStruct((NPAD, DH), jnp.float32),
            jax.ShapeDtypeStruct((NPAD, DH), jnp.float32),
            jax.ShapeDtypeStruct((NPAD, 1), jnp.float32),
        ],
        compiler_params=pltpu.CompilerParams(
            dimension_semantics=("arbitrary",)),
    )(sa, sb, axa, axb, x, alpha, h, wih, whh, bih, bhh, wmat)


def _wmat_body(wr, dr, o):
    dcl = jnp.clip(dr[...], 0.0, 1.0)
    wv = wr[...]
    o[...] = jnp.dot(wv * dcl, wv.T, preferred_element_type=jnp.float32)


def _wmat(w, dvec):
    return pl.pallas_call(
        _wmat_body,
        out_shape=jax.ShapeDtypeStruct((2 * DH, 2 * DH), jnp.float32),
    )(w, dvec.reshape(1, 2 * DH))


def kernel(x, edge_index, edge_weight, rnn_w_ih, rnn_w_hh, rnn_b_ih,
           rnn_b_hh, h, alpha0, w, dvec):
    npad = NPAD - N
    x = jnp.pad(x.astype(jnp.float32), ((0, npad), (0, 0)))
    src = edge_index[1].astype(jnp.int32)
    dst = edge_index[0].astype(jnp.int32)
    ew = edge_weight.astype(jnp.float32)

    order = jnp.argsort(dst)
    ss = src[order]
    dd = dst[order]
    wwb = lax.bitcast_convert_type(ew[order], jnp.int32)

    starts = jnp.searchsorted(dd, jnp.arange(NT, dtype=jnp.int32) * RPT)
    starts = starts.astype(jnp.int32)
    cnt = jnp.diff(jnp.concatenate(
        [starts, jnp.array([E], dtype=jnp.int32)]))
    pcnt = ((cnt + PK - 1) // PK) * PK
    new_start = (jnp.cumsum(pcnt) - pcnt).astype(jnp.int32)
    tile = dd // RPT
    pos = new_start[tile] + (jnp.arange(E, dtype=jnp.int32) - starts[tile])

    srcf = jnp.zeros((TOT,), jnp.int32).at[pos].set(ss)
    wbf = jnp.zeros((TOT,), jnp.int32).at[pos].set(wwb)
    bounds = jnp.concatenate([new_start, jnp.array([TOT], jnp.int32)])
    slot_tile = jnp.clip(
        jnp.searchsorted(bounds, jnp.arange(TOT, dtype=jnp.int32),
                         side='right') - 1, 0, NT - 1).astype(jnp.int32)
    dstf = (slot_tile * RPT).at[pos].set(dd)

    blk = jnp.stack([srcf.reshape(TOTCH, K), dstf.reshape(TOTCH, K),
                     wbf.reshape(TOTCH, K)], axis=1)
    edg = jnp.pad(blk, ((0, 0), (0, 5), (0, 0))).reshape(TOTCH * 8, K)
    meta = jnp.zeros((16, 16), jnp.int32)
    meta = meta.at[:, 0].set((pcnt // K).astype(jnp.int32))
    meta = meta.at[:, 1].set(new_start // K)

    wih = rnn_w_ih.astype(jnp.float32)
    whh = rnn_w_hh.astype(jnp.float32)
    bih = rnn_b_ih.astype(jnp.float32).reshape(1, 2)
    bhh = rnn_b_hh.astype(jnp.float32).reshape(1, 2)
    hf = jnp.pad(h.astype(jnp.float32), ((0, npad), (0, 0)))
    wmat = _wmat(w.astype(jnp.float32), dvec.astype(jnp.float32))

    sa0 = x
    sb0 = jnp.zeros_like(x)
    al0 = jnp.pad(alpha0.astype(jnp.float32), (0, npad)).reshape(NPAD, 1)

    def step(_, carry):
        sa, sb, al = carry
        axa, axb = _spmm(sa, sb, edg, meta)
        sa, sb, al = _update(sa, sb, axa, axb, x, al, hf,
                             wih, whh, bih, bhh, wmat)
        return (sa, sb, al)

    sa, sb, al = lax.fori_loop(0, NSTEPS, step, (sa0, sb0, al0))
    return sa[:N]


# speedup vs baseline: 1.0094x; 1.0094x over previous
import functools

import jax
import jax.numpy as jnp
from jax import lax
from jax.experimental import pallas as pl
from jax.experimental.pallas import tpu as pltpu
from jax.experimental.pallas import tpu_sc as plsc

N = 10000
DH = 128
E = 320000
NSTEPS = 20
DT = 0.9 / NSTEPS

NT = 16
K = 128
NCH = 160
EPT = NCH * K
EPAD = NT * EPT
NPAD = 10240
RPT = NPAD // NT
PK = 2 * K
TOT = E + NT * PK
TOTCH = TOT // K


def _spmm_body(sa, sb, edg, meta, axa, axb,
               acc, ebuf0, ebuf1, rows0, rows1, metab,
               esem0, esem1, gsem0, gsem1):
    c = lax.axis_index("c")
    s = lax.axis_index("s")
    ebuf = (ebuf0, ebuf1)
    rows = (rows0, rows1)
    esem = (esem0, esem1)
    gsem = (gsem0, gsem1)
    base = s * RPT

    pltpu.sync_copy(meta, metab)
    mv = metab[s, pl.ds(0, 16)]
    n = mv[0]
    c0 = mv[1]

    def zacc(r, carry):
        for l in range(DH // 16):
            acc[r, pl.ds(l * 16, 16)] = jnp.zeros((16,), jnp.float32)
        return carry
    lax.fori_loop(0, RPT, zacc, 0)

    def start_stage(it, q):
        off = pl.multiple_of((c0 + it) * (8 * K), 8)
        pltpu.async_copy(edg.at[pl.ds(off, 3 * K)], ebuf[q], esem[q])

    def wait_stage(q):
        pltpu.make_async_copy(edg.at[pl.ds(0, 3 * K)], ebuf[q],
                              esem[q]).wait()

    def start_gather(q):
        idx = ebuf[q].at[pl.ds(0, K)]

        @pl.when(c == 0)
        def _():
            pltpu.async_copy(sa.at[idx], rows[q], gsem[q])

        @pl.when(c == 1)
        def _():
            pltpu.async_copy(sb.at[idx], rows[q], gsem[q])

    def wait_gather(q):
        pltpu.make_async_copy(sa.at[ebuf[0].at[pl.ds(0, K)]], rows[q],
                              gsem[q]).wait()

    def compute(q):
        rq = rows[q]
        eq = ebuf[q]

        def body(jo, cc):
            dv = eq[pl.ds(K + jo * 16, 16)]
            wv = lax.bitcast_convert_type(
                eq[pl.ds(2 * K + jo * 16, 16)], jnp.float32)
            for j in range(16):
                e = jo * 16 + j
                d = dv[j] - base
                wspl = lax.gather(
                    wv, jnp.full((16, 1), j, jnp.int32),
                    lax.GatherDimensionNumbers(
                        offset_dims=(), collapsed_slice_dims=(0,),
                        start_index_map=(0,)),
                    (1,), mode=lax.GatherScatterMode.PROMISE_IN_BOUNDS)
                for l in range(DH // 16):
                    plsc.addupdate(acc.at[d, pl.ds(l * 16, 16)],
                                   rq[e, pl.ds(l * 16, 16)] * wspl)
            return cc
        lax.fori_loop(0, K // 16, body, 0)

    @pl.when(n > 0)
    def _():
        start_stage(0, 0)
        start_stage(1, 1)
        wait_stage(0)
        start_gather(0)

    def pair(p, carry):
        for q in range(2):
            it = 2 * p + q
            q2 = (q + 1) % 2
            wait_gather(q)

            @pl.when(it + 1 < n)
            def _():
                wait_stage(q2)
                start_gather(q2)
            compute(q)

            @pl.when(it + 2 < n)
            def _():
                start_stage(it + 2, q)
        return carry
    lax.fori_loop(0, n // 2, pair, 0)

    @pl.when(c == 0)
    def _():
        pltpu.sync_copy(acc, axa.at[pl.ds(base, RPT)])

    @pl.when(c == 1)
    def _():
        pltpu.sync_copy(acc, axb.at[pl.ds(base, RPT)])


_spmm = functools.partial(
    pl.kernel,
    out_type=(jax.ShapeDtypeStruct((NPAD, DH), jnp.float32),
              jax.ShapeDtypeStruct((NPAD, DH), jnp.float32)),
    mesh=plsc.VectorSubcoreMesh(core_axis_name="c", subcore_axis_name="s"),
    scratch_types=[
        pltpu.VMEM((RPT, DH), jnp.float32),
        pltpu.VMEM((3 * K,), jnp.int32),
        pltpu.VMEM((3 * K,), jnp.int32),
        pltpu.VMEM((K, DH), jnp.float32),
        pltpu.VMEM((K, DH), jnp.float32),
        pltpu.VMEM((16, 16), jnp.int32),
        pltpu.SemaphoreType.DMA,
        pltpu.SemaphoreType.DMA,
        pltpu.SemaphoreType.DMA,
        pltpu.SemaphoreType.DMA,
    ],
)(_spmm_body)


BN = 1024
GRID = NPAD // BN


def _update_body(sa, sb, axa, axb, xr, al, hr, wih, whh, bih, bhh, wmat,
                 osa, osb, oal):
    st = jnp.concatenate([sa[...], sb[...]], axis=1)
    z = jnp.dot(st, wih[...].T, preferred_element_type=jnp.float32)
    z = z + jnp.dot(hr[...], whh[...].T, preferred_element_type=jnp.float32)
    z = z + bih[...] + bhh[...]
    r = jnp.tanh(z)
    alpha_new = al[...] * r[:, 0:1] + r[:, 1:2]
    alph = jax.nn.sigmoid(alpha_new)
    xw = jnp.dot(st, wmat[...], preferred_element_type=jnp.float32)
    ax = jnp.concatenate([axa[...], axb[...]], axis=1)
    xv = xr[...]
    x0 = jnp.concatenate([xv, jnp.zeros_like(xv)], axis=1)
    f = alph * 0.5 * (ax - st) + xw - st + x0
    st2 = st + DT * f
    osa[...] = st2[:, :DH]
    osb[...] = st2[:, DH:]
    oal[...] = alpha_new


def _update(sa, sb, axa, axb, x, alpha, h, wih, whh, bih, bhh, wmat):
    row_spec = pl.BlockSpec((BN, DH), lambda i: (i, 0))
    return pl.pallas_call(
        _update_body,
        grid=(GRID,),
        in_specs=[
            row_spec, row_spec, row_spec, row_spec, row_spec,
            pl.BlockSpec((BN, 1), lambda i: (i, 0)),
            pl.BlockSpec((BN, 2), lambda i: (i, 0)),
            pl.BlockSpec((2, 2 * DH), lambda i: (0, 0)),
            pl.BlockSpec((2, 2), lambda i: (0, 0)),
            pl.BlockSpec((1, 2), lambda i: (0, 0)),
            pl.BlockSpec((1, 2), lambda i: (0, 0)),
            pl.BlockSpec((2 * DH, 2 * DH), lambda i: (0, 0)),
        ],
        out_specs=[
            row_spec, row_spec,
            pl.BlockSpec((BN, 1), lambda i: (i, 0)),
        ],
        out_shape=[
            jax.ShapeDtypeStruct((NPAD, DH), jnp.float32),
            jax.ShapeDtypeStruct((NPAD, DH), jnp.float32),
            jax.ShapeDtypeStruct((NPAD, 1), jnp.float32),
        ],
        compiler_params=pltpu.CompilerParams(
            dimension_semantics=("arbitrary",)),
    )(sa, sb, axa, axb, x, alpha, h, wih, whh, bih, bhh, wmat)


def _wmat_body(wr, dr, o):
    dcl = jnp.clip(dr[...], 0.0, 1.0)
    wv = wr[...]
    o[...] = jnp.dot(wv * dcl, wv.T, preferred_element_type=jnp.float32)


def _wmat(w, dvec):
    return pl.pallas_call(
        _wmat_body,
        out_shape=jax.ShapeDtypeStruct((2 * DH, 2 * DH), jnp.float32),
    )(w, dvec.reshape(1, 2 * DH))


def kernel(x, edge_index, edge_weight, rnn_w_ih, rnn_w_hh, rnn_b_ih,
           rnn_b_hh, h, alpha0, w, dvec):
    npad = NPAD - N
    x = jnp.pad(x.astype(jnp.float32), ((0, npad), (0, 0)))
    src = edge_index[1].astype(jnp.int32)
    dst = edge_index[0].astype(jnp.int32)
    ew = edge_weight.astype(jnp.float32)

    order = jnp.argsort(dst)
    ss = src[order]
    dd = dst[order]
    wwb = lax.bitcast_convert_type(ew[order], jnp.int32)

    starts = jnp.searchsorted(dd, jnp.arange(NT, dtype=jnp.int32) * RPT)
    starts = starts.astype(jnp.int32)
    cnt = jnp.diff(jnp.concatenate(
        [starts, jnp.array([E], dtype=jnp.int32)]))
    pcnt = ((cnt + PK - 1) // PK) * PK
    new_start = (jnp.cumsum(pcnt) - pcnt).astype(jnp.int32)
    tile = dd // RPT
    pos = new_start[tile] + (jnp.arange(E, dtype=jnp.int32) - starts[tile])

    srcf = jnp.zeros((TOT,), jnp.int32).at[pos].set(ss)
    wbf = jnp.zeros((TOT,), jnp.int32).at[pos].set(wwb)
    bounds = jnp.concatenate([new_start, jnp.array([TOT], jnp.int32)])
    slot_tile = jnp.clip(
        jnp.searchsorted(bounds, jnp.arange(TOT, dtype=jnp.int32),
                         side='right') - 1, 0, NT - 1).astype(jnp.int32)
    dstf = (slot_tile * RPT).at[pos].set(dd)

    blk = jnp.stack([srcf.reshape(TOTCH, K), dstf.reshape(TOTCH, K),
                     wbf.reshape(TOTCH, K)], axis=1)
    edg = jnp.pad(blk, ((0, 0), (0, 5), (0, 0))).reshape(TOTCH * 8 * K)
    meta = jnp.zeros((16, 16), jnp.int32)
    meta = meta.at[:, 0].set((pcnt // K).astype(jnp.int32))
    meta = meta.at[:, 1].set(new_start // K)

    wih = rnn_w_ih.astype(jnp.float32)
    whh = rnn_w_hh.astype(jnp.float32)
    bih = rnn_b_ih.astype(jnp.float32).reshape(1, 2)
    bhh = rnn_b_hh.astype(jnp.float32).reshape(1, 2)
    hf = jnp.pad(h.astype(jnp.float32), ((0, npad), (0, 0)))
    wmat = _wmat(w.astype(jnp.float32), dvec.astype(jnp.float32))

    sa0 = x
    sb0 = jnp.zeros_like(x)
    al0 = jnp.pad(alpha0.astype(jnp.float32), (0, npad)).reshape(NPAD, 1)

    def step(_, carry):
        sa, sb, al = carry
        axa, axb = _spmm(sa, sb, edg, meta)
        sa, sb, al = _update(sa, sb, axa, axb, x, al, hf,
                             wih, whh, bih, bhh, wmat)
        return (sa, sb, al)

    sa, sb, al = lax.fori_loop(0, NSTEPS, step, (sa0, sb0, al0))
    return sa[:N]


# speedup vs baseline: 1.5923x; 1.5775x over previous
import functools

import jax
import jax.numpy as jnp
from jax import lax
from jax.experimental import pallas as pl
from jax.experimental.pallas import tpu as pltpu
from jax.experimental.pallas import tpu_sc as plsc

N = 10000
DH = 128
E = 320000
NSTEPS = 20
DT = 0.9 / NSTEPS

NT = 16
K = 128
NCH = 160
EPT = NCH * K
EPAD = NT * EPT
NPAD = 10240
RPT = NPAD // NT
SCH = 16
NCHS = NCH // SCH


def _spmm_body(sa, sb, edg, axa, axb,
               acc, ebuf, rows0, rows1,
               gsem0, gsem1, ssem0, ssem1):
    c = lax.axis_index("c")
    s = lax.axis_index("s")
    rows = (rows0, rows1)
    gsem = (gsem0, gsem1)
    ssem = (ssem0, ssem1)

    def zrow(e, carry):
        for l in range(DH // 16):
            rows0[e, pl.ds(l * 16, 16)] = jnp.zeros((16,), jnp.float32)
        return carry
    lax.fori_loop(0, K, zrow, 0)

    base = s * RPT
    for j in range(RPT // K):
        pltpu.sync_copy(rows0, acc.at[pl.ds(base + j * K, K)])
    rem = RPT % K
    if rem:
        pltpu.sync_copy(rows0.at[pl.ds(0, rem)],
                        acc.at[pl.ds(base + (RPT // K) * K, rem)])
    plsc.subcore_barrier()

    def superchunk(sc, carry):
        row0 = (s * NCHS + sc) * (3 * SCH)
        pltpu.sync_copy(edg.at[pl.ds(row0, 3 * SCH)], ebuf)

        def start_gather(it, b):
            @pl.when(c == 0)
            def _():
                pltpu.async_copy(sa.at[ebuf.at[it]], rows[b], gsem[b])

            @pl.when(c == 1)
            def _():
                pltpu.async_copy(sb.at[ebuf.at[it]], rows[b], gsem[b])

        def wait_gather(b):
            pltpu.make_async_copy(sa.at[ebuf.at[0]], rows[b], gsem[b]).wait()

        def scale(it, b):
            rb = rows[b]

            def body(jo, c2):
                wv = lax.bitcast_convert_type(
                    ebuf[2 * SCH + it, pl.ds(jo * 16, 16)], jnp.float32)
                for j in range(16):
                    e = jo * 16 + j
                    wspl = lax.gather(
                        wv, jnp.full((16, 1), j, jnp.int32),
                        lax.GatherDimensionNumbers(
                            offset_dims=(), collapsed_slice_dims=(0,),
                            start_index_map=(0,)),
                        (1,), mode=lax.GatherScatterMode.PROMISE_IN_BOUNDS)
                    for l in range(DH // 16):
                        rb[e, pl.ds(l * 16, 16)] = (
                            rb[e, pl.ds(l * 16, 16)] * wspl)
                return c2
            lax.fori_loop(0, K // 16, body, 0)

        scatters = {}
        start_gather(0, 0)
        for it in range(SCH):
            b = it % 2
            if it + 1 < SCH:
                b2 = (it + 1) % 2
                if it - 1 >= 0:
                    scatters[it - 1].wait()
                start_gather(it + 1, b2)
            wait_gather(b)
            scale(it, b)
            scatters[it] = pltpu.async_copy(
                rows[b], acc.at[ebuf.at[SCH + it]], ssem[b], add=True)
        scatters[SCH - 2].wait()
        scatters[SCH - 1].wait()
        return carry
    lax.fori_loop(0, NCHS, superchunk, 0)

    plsc.subcore_barrier()

    @pl.when(c == 0)
    def _():
        pltpu.sync_copy(acc.at[pl.ds(base, RPT)], axa.at[pl.ds(base, RPT)])

    @pl.when(c == 1)
    def _():
        pltpu.sync_copy(acc.at[pl.ds(base, RPT)], axb.at[pl.ds(base, RPT)])


_spmm = functools.partial(
    pl.kernel,
    out_type=(jax.ShapeDtypeStruct((NPAD, DH), jnp.float32),
              jax.ShapeDtypeStruct((NPAD, DH), jnp.float32)),
    mesh=plsc.VectorSubcoreMesh(core_axis_name="c", subcore_axis_name="s"),
    scratch_types=[
        pltpu.VMEM_SHARED((NPAD, DH), jnp.float32),
        pltpu.VMEM((3 * SCH, K), jnp.int32),
        pltpu.VMEM((K, DH), jnp.float32),
        pltpu.VMEM((K, DH), jnp.float32),
        pltpu.SemaphoreType.DMA,
        pltpu.SemaphoreType.DMA,
        pltpu.SemaphoreType.DMA,
        pltpu.SemaphoreType.DMA,
    ],
)(_spmm_body)


BN = 1024
GRID = NPAD // BN


def _update_body(sa, sb, axa, axb, xr, al, hr, wih, whh, bih, bhh, wmat,
                 osa, osb, oal):
    st = jnp.concatenate([sa[...], sb[...]], axis=1)
    z = jnp.dot(st, wih[...].T, preferred_element_type=jnp.float32)
    z = z + jnp.dot(hr[...], whh[...].T, preferred_element_type=jnp.float32)
    z = z + bih[...] + bhh[...]
    r = jnp.tanh(z)
    alpha_new = al[...] * r[:, 0:1] + r[:, 1:2]
    alph = jax.nn.sigmoid(alpha_new)
    xw = jnp.dot(st, wmat[...], preferred_element_type=jnp.float32)
    ax = jnp.concatenate([axa[...], axb[...]], axis=1)
    xv = xr[...]
    x0 = jnp.concatenate([xv, jnp.zeros_like(xv)], axis=1)
    f = alph * 0.5 * (ax - st) + xw - st + x0
    st2 = st + DT * f
    osa[...] = st2[:, :DH]
    osb[...] = st2[:, DH:]
    oal[...] = alpha_new


def _update(sa, sb, axa, axb, x, alpha, h, wih, whh, bih, bhh, wmat):
    row_spec = pl.BlockSpec((BN, DH), lambda i: (i, 0))
    return pl.pallas_call(
        _update_body,
        grid=(GRID,),
        in_specs=[
            row_spec, row_spec, row_spec, row_spec, row_spec,
            pl.BlockSpec((BN, 1), lambda i: (i, 0)),
            pl.BlockSpec((BN, 2), lambda i: (i, 0)),
            pl.BlockSpec((2, 2 * DH), lambda i: (0, 0)),
            pl.BlockSpec((2, 2), lambda i: (0, 0)),
            pl.BlockSpec((1, 2), lambda i: (0, 0)),
            pl.BlockSpec((1, 2), lambda i: (0, 0)),
            pl.BlockSpec((2 * DH, 2 * DH), lambda i: (0, 0)),
        ],
        out_specs=[
            row_spec, row_spec,
            pl.BlockSpec((BN, 1), lambda i: (i, 0)),
        ],
        out_shape=[
            jax.ShapeDtypeStruct((NPAD, DH), jnp.float32),
            jax.ShapeDtypeStruct((NPAD, DH), jnp.float32),
            jax.ShapeDtypeStruct((NPAD, 1), jnp.float32),
        ],
        compiler_params=pltpu.CompilerParams(
            dimension_semantics=("arbitrary",)),
    )(sa, sb, axa, axb, x, alpha, h, wih, whh, bih, bhh, wmat)


def _wmat_body(wr, dr, o):
    dcl = jnp.clip(dr[...], 0.0, 1.0)
    wv = wr[...]
    o[...] = jnp.dot(wv * dcl, wv.T, preferred_element_type=jnp.float32)


def _wmat(w, dvec):
    return pl.pallas_call(
        _wmat_body,
        out_shape=jax.ShapeDtypeStruct((2 * DH, 2 * DH), jnp.float32),
    )(w, dvec.reshape(1, 2 * DH))


def kernel(x, edge_index, edge_weight, rnn_w_ih, rnn_w_hh, rnn_b_ih,
           rnn_b_hh, h, alpha0, w, dvec):
    npad = NPAD - N
    x = jnp.pad(x.astype(jnp.float32), ((0, npad), (0, 0)))
    src = edge_index[1].astype(jnp.int32)
    dst = edge_index[0].astype(jnp.int32)
    ew = edge_weight.astype(jnp.float32)

    order = jnp.argsort(dst)
    src = src[order]
    dst = dst[order]
    ew = ew[order]

    pad = EPAD - E
    srcp = jnp.pad(src, (0, pad)).reshape(NT, NCHS, SCH, K)
    dstp = jnp.pad(dst, (0, pad)).reshape(NT, NCHS, SCH, K)
    wbits = lax.bitcast_convert_type(jnp.pad(ew, (0, pad)), jnp.int32)
    wp = wbits.reshape(NT, NCHS, SCH, K)
    edg = jnp.stack([srcp, dstp, wp], axis=2).reshape(NT * NCHS * 3 * SCH, K)

    wih = rnn_w_ih.astype(jnp.float32)
    whh = rnn_w_hh.astype(jnp.float32)
    bih = rnn_b_ih.astype(jnp.float32).reshape(1, 2)
    bhh = rnn_b_hh.astype(jnp.float32).reshape(1, 2)
    hf = jnp.pad(h.astype(jnp.float32), ((0, npad), (0, 0)))
    wmat = _wmat(w.astype(jnp.float32), dvec.astype(jnp.float32))

    sa0 = x
    sb0 = jnp.zeros_like(x)
    al0 = jnp.pad(alpha0.astype(jnp.float32), (0, npad)).reshape(NPAD, 1)

    def step(_, carry):
        sa, sb, al = carry
        axa, axb = _spmm(sa, sb, edg)
        sa, sb, al = _update(sa, sb, axa, axb, x, al, hf,
                             wih, whh, bih, bhh, wmat)
        return (sa, sb, al)

    sa, sb, al = lax.fori_loop(0, NSTEPS, step, (sa0, sb0, al0))
    return sa[:N]


# speedup vs baseline: 1.6191x; 1.0168x over previous
import functools

import jax
import jax.numpy as jnp
from jax import lax
from jax.experimental import pallas as pl
from jax.experimental.pallas import tpu as pltpu
from jax.experimental.pallas import tpu_sc as plsc

N = 10000
DH = 128
E = 320000
NSTEPS = 20
DT = 0.9 / NSTEPS

NT = 16
K = 128
NCH = 160
EPT = NCH * K
EPAD = NT * EPT
NPAD = 10240
RPT = NPAD // NT
SCH = 32
NCHS = NCH // SCH


def _spmm_body(sa, sb, edg, axa, axb,
               acc, ebuf, rows0, rows1,
               gsem0, gsem1, ssem0, ssem1):
    c = lax.axis_index("c")
    s = lax.axis_index("s")
    rows = (rows0, rows1)
    gsem = (gsem0, gsem1)
    ssem = (ssem0, ssem1)

    def zrow(e, carry):
        for l in range(DH // 16):
            rows0[e, pl.ds(l * 16, 16)] = jnp.zeros((16,), jnp.float32)
        return carry
    lax.fori_loop(0, K, zrow, 0)

    base = s * RPT
    for j in range(RPT // K):
        pltpu.sync_copy(rows0, acc.at[pl.ds(base + j * K, K)])
    rem = RPT % K
    if rem:
        pltpu.sync_copy(rows0.at[pl.ds(0, rem)],
                        acc.at[pl.ds(base + (RPT // K) * K, rem)])
    plsc.subcore_barrier()

    def superchunk(sc, carry):
        row0 = (s * NCHS + sc) * (3 * SCH)
        pltpu.sync_copy(edg.at[pl.ds(row0, 3 * SCH)], ebuf)

        def start_gather(it, b):
            @pl.when(c == 0)
            def _():
                pltpu.async_copy(sa.at[ebuf.at[it]], rows[b], gsem[b])

            @pl.when(c == 1)
            def _():
                pltpu.async_copy(sb.at[ebuf.at[it]], rows[b], gsem[b])

        def wait_gather(b):
            pltpu.make_async_copy(sa.at[ebuf.at[0]], rows[b], gsem[b]).wait()

        def scale(it, b):
            rb = rows[b]

            def body(jo, c2):
                wv = lax.bitcast_convert_type(
                    ebuf[2 * SCH + it, pl.ds(jo * 16, 16)], jnp.float32)
                for j in range(16):
                    e = jo * 16 + j
                    wspl = lax.gather(
                        wv, jnp.full((16, 1), j, jnp.int32),
                        lax.GatherDimensionNumbers(
                            offset_dims=(), collapsed_slice_dims=(0,),
                            start_index_map=(0,)),
                        (1,), mode=lax.GatherScatterMode.PROMISE_IN_BOUNDS)
                    for l in range(DH // 16):
                        rb[e, pl.ds(l * 16, 16)] = (
                            rb[e, pl.ds(l * 16, 16)] * wspl)
                return c2
            lax.fori_loop(0, K // 16, body, 0)

        scatters = {}
        start_gather(0, 0)
        for it in range(SCH):
            b = it % 2
            if it + 1 < SCH:
                b2 = (it + 1) % 2
                if it - 1 >= 0:
                    scatters[it - 1].wait()
                start_gather(it + 1, b2)
            wait_gather(b)
            scale(it, b)
            scatters[it] = pltpu.async_copy(
                rows[b], acc.at[ebuf.at[SCH + it]], ssem[b], add=True)
        scatters[SCH - 2].wait()
        scatters[SCH - 1].wait()
        return carry
    lax.fori_loop(0, NCHS, superchunk, 0)

    plsc.subcore_barrier()

    @pl.when(c == 0)
    def _():
        pltpu.sync_copy(acc.at[pl.ds(base, RPT)], axa.at[pl.ds(base, RPT)])

    @pl.when(c == 1)
    def _():
        pltpu.sync_copy(acc.at[pl.ds(base, RPT)], axb.at[pl.ds(base, RPT)])


_spmm = functools.partial(
    pl.kernel,
    out_type=(jax.ShapeDtypeStruct((NPAD, DH), jnp.float32),
              jax.ShapeDtypeStruct((NPAD, DH), jnp.float32)),
    mesh=plsc.VectorSubcoreMesh(core_axis_name="c", subcore_axis_name="s"),
    scratch_types=[
        pltpu.VMEM_SHARED((NPAD, DH), jnp.float32),
        pltpu.VMEM((3 * SCH, K), jnp.int32),
        pltpu.VMEM((K, DH), jnp.float32),
        pltpu.VMEM((K, DH), jnp.float32),
        pltpu.SemaphoreType.DMA,
        pltpu.SemaphoreType.DMA,
        pltpu.SemaphoreType.DMA,
        pltpu.SemaphoreType.DMA,
    ],
)(_spmm_body)


BN = 1024
GRID = NPAD // BN


def _update_body(sa, sb, axa, axb, xr, al, hr, wih, whh, bih, bhh, wmat,
                 osa, osb, oal):
    st = jnp.concatenate([sa[...], sb[...]], axis=1)
    z = jnp.dot(st, wih[...].T, preferred_element_type=jnp.float32)
    z = z + jnp.dot(hr[...], whh[...].T, preferred_element_type=jnp.float32)
    z = z + bih[...] + bhh[...]
    r = jnp.tanh(z)
    alpha_new = al[...] * r[:, 0:1] + r[:, 1:2]
    alph = jax.nn.sigmoid(alpha_new)
    xw = jnp.dot(st, wmat[...], preferred_element_type=jnp.float32)
    ax = jnp.concatenate([axa[...], axb[...]], axis=1)
    xv = xr[...]
    x0 = jnp.concatenate([xv, jnp.zeros_like(xv)], axis=1)
    f = alph * 0.5 * (ax - st) + xw - st + x0
    st2 = st + DT * f
    osa[...] = st2[:, :DH]
    osb[...] = st2[:, DH:]
    oal[...] = alpha_new


def _update(sa, sb, axa, axb, x, alpha, h, wih, whh, bih, bhh, wmat):
    row_spec = pl.BlockSpec((BN, DH), lambda i: (i, 0))
    return pl.pallas_call(
        _update_body,
        grid=(GRID,),
        in_specs=[
            row_spec, row_spec, row_spec, row_spec, row_spec,
            pl.BlockSpec((BN, 1), lambda i: (i, 0)),
            pl.BlockSpec((BN, 2), lambda i: (i, 0)),
            pl.BlockSpec((2, 2 * DH), lambda i: (0, 0)),
            pl.BlockSpec((2, 2), lambda i: (0, 0)),
            pl.BlockSpec((1, 2), lambda i: (0, 0)),
            pl.BlockSpec((1, 2), lambda i: (0, 0)),
            pl.BlockSpec((2 * DH, 2 * DH), lambda i: (0, 0)),
        ],
        out_specs=[
            row_spec, row_spec,
            pl.BlockSpec((BN, 1), lambda i: (i, 0)),
        ],
        out_shape=[
            jax.ShapeDtypeStruct((NPAD, DH), jnp.float32),
            jax.ShapeDtypeStruct((NPAD, DH), jnp.float32),
            jax.ShapeDtypeStruct((NPAD, 1), jnp.float32),
        ],
        compiler_params=pltpu.CompilerParams(
            dimension_semantics=("arbitrary",)),
    )(sa, sb, axa, axb, x, alpha, h, wih, whh, bih, bhh, wmat)


def _wmat_body(wr, dr, o):
    dcl = jnp.clip(dr[...], 0.0, 1.0)
    wv = wr[...]
    o[...] = jnp.dot(wv * dcl, wv.T, preferred_element_type=jnp.float32)


def _wmat(w, dvec):
    return pl.pallas_call(
        _wmat_body,
        out_shape=jax.ShapeDtypeStruct((2 * DH, 2 * DH), jnp.float32),
    )(w, dvec.reshape(1, 2 * DH))


def kernel(x, edge_index, edge_weight, rnn_w_ih, rnn_w_hh, rnn_b_ih,
           rnn_b_hh, h, alpha0, w, dvec):
    npad = NPAD - N
    x = jnp.pad(x.astype(jnp.float32), ((0, npad), (0, 0)))
    src = edge_index[1].astype(jnp.int32)
    dst = edge_index[0].astype(jnp.int32)
    ew = edge_weight.astype(jnp.float32)

    order = jnp.argsort(dst)
    src = src[order]
    dst = dst[order]
    ew = ew[order]

    pad = EPAD - E
    srcp = jnp.pad(src, (0, pad)).reshape(NT, NCHS, SCH, K)
    dstp = jnp.pad(dst, (0, pad)).reshape(NT, NCHS, SCH, K)
    wbits = lax.bitcast_convert_type(jnp.pad(ew, (0, pad)), jnp.int32)
    wp = wbits.reshape(NT, NCHS, SCH, K)
    edg = jnp.stack([srcp, dstp, wp], axis=2).reshape(NT * NCHS * 3 * SCH, K)

    wih = rnn_w_ih.astype(jnp.float32)
    whh = rnn_w_hh.astype(jnp.float32)
    bih = rnn_b_ih.astype(jnp.float32).reshape(1, 2)
    bhh = rnn_b_hh.astype(jnp.float32).reshape(1, 2)
    hf = jnp.pad(h.astype(jnp.float32), ((0, npad), (0, 0)))
    wmat = _wmat(w.astype(jnp.float32), dvec.astype(jnp.float32))

    sa0 = x
    sb0 = jnp.zeros_like(x)
    al0 = jnp.pad(alpha0.astype(jnp.float32), (0, npad)).reshape(NPAD, 1)

    def step(_, carry):
        sa, sb, al = carry
        axa, axb = _spmm(sa, sb, edg)
        sa, sb, al = _update(sa, sb, axa, axb, x, al, hf,
                             wih, whh, bih, bhh, wmat)
        return (sa, sb, al)

    sa, sb, al = lax.fori_loop(0, NSTEPS, step, (sa0, sb0, al0))
    return sa[:N]


# speedup vs baseline: 1.7288x; 1.0678x over previous
import functools

import jax
import jax.numpy as jnp
from jax import lax
from jax.experimental import pallas as pl
from jax.experimental.pallas import tpu as pltpu
from jax.experimental.pallas import tpu_sc as plsc

N = 10000
DH = 128
E = 320000
NSTEPS = 20
DT = 0.9 / NSTEPS

NT = 16
K = 128
NCH = 160
EPT = NCH * K
EPAD = NT * EPT
NPAD = 10240
RPT = NPAD // NT
SCH = 32
NCHS = NCH // SCH


def _spmm_body(sa, sb, edg, axa, axb,
               acc, ebuf, rows0, rows1,
               gsem0, gsem1, ssem0, ssem1):
    c = lax.axis_index("c")
    s = lax.axis_index("s")
    rows = (rows0, rows1)
    gsem = (gsem0, gsem1)
    ssem = (ssem0, ssem1)

    def zrow(e, carry):
        for l in range(DH // 16):
            rows0[e, pl.ds(l * 16, 16)] = jnp.zeros((16,), jnp.float32)
        return carry
    lax.fori_loop(0, K, zrow, 0)

    base = s * RPT
    for j in range(RPT // K):
        pltpu.sync_copy(rows0, acc.at[pl.ds(base + j * K, K)])
    rem = RPT % K
    if rem:
        pltpu.sync_copy(rows0.at[pl.ds(0, rem)],
                        acc.at[pl.ds(base + (RPT // K) * K, rem)])
    plsc.subcore_barrier()

    def superchunk(sc, carry):
        row0 = (s * NCHS + sc) * (3 * SCH)
        pltpu.sync_copy(edg.at[pl.ds(row0, 3 * SCH)], ebuf)

        def start_gather(it, b):
            @pl.when(c == 0)
            def _():
                pltpu.async_copy(sa.at[ebuf.at[it]], rows[b], gsem[b])

            @pl.when(c == 1)
            def _():
                pltpu.async_copy(sb.at[ebuf.at[it]], rows[b], gsem[b])

        def wait_gather(b):
            pltpu.make_async_copy(sa.at[ebuf.at[0]], rows[b], gsem[b]).wait()

        def scale(it, b):
            rb = rows[b]

            def body(jo, c2):
                wv = lax.bitcast_convert_type(
                    ebuf[2 * SCH + it, pl.ds(jo * 16, 16)], jnp.float32)
                for j in range(16):
                    e = jo * 16 + j
                    wspl = lax.gather(
                        wv, jnp.full((16, 1), j, jnp.int32),
                        lax.GatherDimensionNumbers(
                            offset_dims=(), collapsed_slice_dims=(0,),
                            start_index_map=(0,)),
                        (1,), mode=lax.GatherScatterMode.PROMISE_IN_BOUNDS)
                    for l in range(DH // 16):
                        rb[e, pl.ds(l * 16, 16)] = (
                            rb[e, pl.ds(l * 16, 16)] * wspl)
                return c2
            lax.fori_loop(0, K // 16, body, 0)

        scatters = {}
        start_gather(0, 0)
        for it in range(SCH):
            b = it % 2
            if it + 1 < SCH:
                b2 = (it + 1) % 2
                if it - 1 >= 0:
                    scatters[it - 1].wait()
                start_gather(it + 1, b2)
            wait_gather(b)
            scale(it, b)
            scatters[it] = pltpu.async_copy(
                rows[b], acc.at[ebuf.at[SCH + it]], ssem[b], add=True)
        scatters[SCH - 2].wait()
        scatters[SCH - 1].wait()
        return carry
    lax.fori_loop(0, NCHS, superchunk, 0)

    plsc.subcore_barrier()

    @pl.when(c == 0)
    def _():
        pltpu.sync_copy(acc.at[pl.ds(base, RPT)], axa.at[pl.ds(base, RPT)])

    @pl.when(c == 1)
    def _():
        pltpu.sync_copy(acc.at[pl.ds(base, RPT)], axb.at[pl.ds(base, RPT)])


_spmm = functools.partial(
    pl.kernel,
    out_type=(jax.ShapeDtypeStruct((NPAD, DH), jnp.float32),
              jax.ShapeDtypeStruct((NPAD, DH), jnp.float32)),
    mesh=plsc.VectorSubcoreMesh(core_axis_name="c", subcore_axis_name="s"),
    scratch_types=[
        pltpu.VMEM_SHARED((NPAD, DH), jnp.float32),
        pltpu.VMEM((3 * SCH, K), jnp.int32),
        pltpu.VMEM((K, DH), jnp.float32),
        pltpu.VMEM((K, DH), jnp.float32),
        pltpu.SemaphoreType.DMA,
        pltpu.SemaphoreType.DMA,
        pltpu.SemaphoreType.DMA,
        pltpu.SemaphoreType.DMA,
    ],
)(_spmm_body)


BN = 1024
GRID = NPAD // BN


def _update_pre_body(sa, sb, xr, al, hr, wih, whh, bih, bhh, wmat,
                     opa, opb, oal, oalph):
    st = jnp.concatenate([sa[...], sb[...]], axis=1)
    z = jnp.dot(st, wih[...].T, preferred_element_type=jnp.float32)
    z = z + jnp.dot(hr[...], whh[...].T, preferred_element_type=jnp.float32)
    z = z + bih[...] + bhh[...]
    r = jnp.tanh(z)
    alpha_new = al[...] * r[:, 0:1] + r[:, 1:2]
    alph = jax.nn.sigmoid(alpha_new)
    xw = jnp.dot(st, wmat[...], preferred_element_type=jnp.float32)
    xv = xr[...]
    x0 = jnp.concatenate([xv, jnp.zeros_like(xv)], axis=1)
    pre = st + DT * (xw - st + x0) - (DT * 0.5) * alph * st
    opa[...] = pre[:, :DH]
    opb[...] = pre[:, DH:]
    oal[...] = alpha_new
    oalph[...] = alph


def _update_pre(sa, sb, x, alpha, h, wih, whh, bih, bhh, wmat):
    row_spec = pl.BlockSpec((BN, DH), lambda i: (i, 0))
    col_spec = pl.BlockSpec((BN, 1), lambda i: (i, 0))
    return pl.pallas_call(
        _update_pre_body,
        grid=(GRID,),
        in_specs=[
            row_spec, row_spec, row_spec,
            col_spec,
            pl.BlockSpec((BN, 2), lambda i: (i, 0)),
            pl.BlockSpec((2, 2 * DH), lambda i: (0, 0)),
            pl.BlockSpec((2, 2), lambda i: (0, 0)),
            pl.BlockSpec((1, 2), lambda i: (0, 0)),
            pl.BlockSpec((1, 2), lambda i: (0, 0)),
            pl.BlockSpec((2 * DH, 2 * DH), lambda i: (0, 0)),
        ],
        out_specs=[row_spec, row_spec, col_spec, col_spec],
        out_shape=[
            jax.ShapeDtypeStruct((NPAD, DH), jnp.float32),
            jax.ShapeDtypeStruct((NPAD, DH), jnp.float32),
            jax.ShapeDtypeStruct((NPAD, 1), jnp.float32),
            jax.ShapeDtypeStruct((NPAD, 1), jnp.float32),
        ],
        compiler_params=pltpu.CompilerParams(
            dimension_semantics=("arbitrary",)),
    )(sa, sb, x, alpha, h, wih, whh, bih, bhh, wmat)


def _update_post_body(pa, pb, axa, axb, alph, osa, osb):
    k = (DT * 0.5) * alph[...]
    osa[...] = pa[...] + k * axa[...]
    osb[...] = pb[...] + k * axb[...]


def _update_post(pa, pb, axa, axb, alph):
    row_spec = pl.BlockSpec((BN, DH), lambda i: (i, 0))
    return pl.pallas_call(
        _update_post_body,
        grid=(GRID,),
        in_specs=[row_spec, row_spec, row_spec, row_spec,
                  pl.BlockSpec((BN, 1), lambda i: (i, 0))],
        out_specs=[row_spec, row_spec],
        out_shape=[
            jax.ShapeDtypeStruct((NPAD, DH), jnp.float32),
            jax.ShapeDtypeStruct((NPAD, DH), jnp.float32),
        ],
        compiler_params=pltpu.CompilerParams(
            dimension_semantics=("arbitrary",)),
    )(pa, pb, axa, axb, alph)


def _wmat_body(wr, dr, o):
    dcl = jnp.clip(dr[...], 0.0, 1.0)
    wv = wr[...]
    o[...] = jnp.dot(wv * dcl, wv.T, preferred_element_type=jnp.float32)


def _wmat(w, dvec):
    return pl.pallas_call(
        _wmat_body,
        out_shape=jax.ShapeDtypeStruct((2 * DH, 2 * DH), jnp.float32),
    )(w, dvec.reshape(1, 2 * DH))


def kernel(x, edge_index, edge_weight, rnn_w_ih, rnn_w_hh, rnn_b_ih,
           rnn_b_hh, h, alpha0, w, dvec):
    npad = NPAD - N
    x = jnp.pad(x.astype(jnp.float32), ((0, npad), (0, 0)))
    src = edge_index[1].astype(jnp.int32)
    dst = edge_index[0].astype(jnp.int32)
    ew = edge_weight.astype(jnp.float32)

    order = jnp.argsort(dst)
    src = src[order]
    dst = dst[order]
    ew = ew[order]

    pad = EPAD - E
    srcp = jnp.pad(src, (0, pad)).reshape(NT, NCHS, SCH, K)
    dstp = jnp.pad(dst, (0, pad)).reshape(NT, NCHS, SCH, K)
    wbits = lax.bitcast_convert_type(jnp.pad(ew, (0, pad)), jnp.int32)
    wp = wbits.reshape(NT, NCHS, SCH, K)
    edg = jnp.stack([srcp, dstp, wp], axis=2).reshape(NT * NCHS * 3 * SCH, K)

    wih = rnn_w_ih.astype(jnp.float32)
    whh = rnn_w_hh.astype(jnp.float32)
    bih = rnn_b_ih.astype(jnp.float32).reshape(1, 2)
    bhh = rnn_b_hh.astype(jnp.float32).reshape(1, 2)
    hf = jnp.pad(h.astype(jnp.float32), ((0, npad), (0, 0)))
    wmat = _wmat(w.astype(jnp.float32), dvec.astype(jnp.float32))

    sa0 = x
    sb0 = jnp.zeros_like(x)
    al0 = jnp.pad(alpha0.astype(jnp.float32), (0, npad)).reshape(NPAD, 1)

    def step(_, carry):
        sa, sb, al = carry
        axa, axb = _spmm(sa, sb, edg)
        pa, pb, al, alph = _update_pre(sa, sb, x, al, hf,
                                       wih, whh, bih, bhh, wmat)
        sa, sb = _update_post(pa, pb, axa, axb, alph)
        return (sa, sb, al)

    sa, sb, al = lax.fori_loop(0, NSTEPS, step, (sa0, sb0, al0))
    return sa[:N]
